# baseline (speedup 1.0000x reference)
import functools

import jax
import jax.numpy as jnp
from jax import lax
from jax.experimental import pallas as pl
from jax.experimental.pallas import tpu as pltpu
from jax.experimental.pallas import tpu_sc as plsc

NC = 2
NS = 16
NW = NC * NS

W = 72
NBUF = 3


def _sc_gather(table, idx):
    b_total = idx.shape[0]
    d = table.shape[1]
    assert b_total % NW == 0
    b_per_w = b_total // NW
    n_full = b_per_w // W
    tail = b_per_w - n_full * W
    n_rounds = n_full // NBUF
    assert n_full % NBUF == 0 and tail % 8 == 0

    mesh = plsc.VectorSubcoreMesh(core_axis_name="c", subcore_axis_name="s")

    @functools.partial(
        pl.kernel,
        mesh=mesh,
        out_type=jax.ShapeDtypeStruct((b_total, d), table.dtype),
        scratch_types=[
            pltpu.VMEM((b_per_w,), jnp.int32),
        ] + [pltpu.VMEM((W, d), table.dtype) for _ in range(NBUF)]
          + [pltpu.VMEM((tail, d), table.dtype)]
          + [pltpu.SemaphoreType.DMA for _ in range(2 * NBUF)],
    )
    def k(table_hbm, idx_hbm, out_hbm, idx_v, *bufs_and_sems):
        bufs = bufs_and_sems[:NBUF]
        tbuf = bufs_and_sems[NBUF]
        gs = bufs_and_sems[NBUF + 1:2 * NBUF + 1]
        os = bufs_and_sems[2 * NBUF + 1:]
        wid = lax.axis_index("s") * NC + lax.axis_index("c")
        base = wid * b_per_w
        pltpu.sync_copy(idx_hbm.at[pl.ds(base, b_per_w)], idx_v)

        def gather(win, b):
            return pltpu.make_async_copy(
                table_hbm.at[idx_v.at[pl.ds(win * W, W)]], bufs[b], gs[b])

        def drain(win, b):
            return pltpu.make_async_copy(
                bufs[b], out_hbm.at[pl.ds(base + win * W, W)], os[b])

        for b in range(NBUF):
            gather(b, b).start()

        @pl.loop(0, n_rounds)
        def _(r):
            w0 = NBUF * r
            for b in range(NBUF):
                gather(w0 + b, b).wait()
                drain(w0 + b, b).start()

            @pl.when(r < n_rounds - 1)
            def _():
                for b in range(NBUF):
                    drain(w0 + b, b).wait()
                    gather(w0 + NBUF + b, b).start()

        t_off = n_full * W
        tgather = pltpu.make_async_copy(
            table_hbm.at[idx_v.at[pl.ds(t_off, tail)]], tbuf, gs[0])
        tgather.start()

        for b in range(NBUF):
            drain(0, b).wait()

        tgather.wait()
        pltpu.sync_copy(tbuf, out_hbm.at[pl.ds(base + t_off, tail)])

    return k(table, idx)


def kernel(x, embed_weight):
    b, t = x.shape
    d = embed_weight.shape[1]
    flat = x.T.reshape(b * t).astype(jnp.int32)
    out = _sc_gather(embed_weight, flat)
    return out.reshape(t, b, d).transpose(1, 0, 2)


# speedup vs baseline: 1.0094x; 1.0094x over previous
import functools

import jax
import jax.numpy as jnp
from jax import lax
from jax.experimental import pallas as pl
from jax.experimental.pallas import tpu as pltpu
from jax.experimental.pallas import tpu_sc as plsc

NC = 2
NS = 16
NW = NC * NS

W = 72
NBUF = 3


def _sc_gather(table, idx):
    b_total = idx.shape[0]
    d = table.shape[1]
    assert b_total % NW == 0
    b_per_w = b_total // NW
    n_full = b_per_w // W
    tail = b_per_w - n_full * W
    n_rounds = n_full // NBUF
    assert n_full % NBUF == 0 and tail % 8 == 0 and tail > 0

    mesh = plsc.VectorSubcoreMesh(core_axis_name="c", subcore_axis_name="s")

    @functools.partial(
        pl.kernel,
        mesh=mesh,
        out_type=jax.ShapeDtypeStruct((b_total, d), table.dtype),
        scratch_types=[
            pltpu.VMEM((b_per_w,), jnp.int32),
        ] + [pltpu.VMEM((W, d), table.dtype) for _ in range(NBUF)]
          + [pltpu.VMEM((tail, d), table.dtype)]
          + [pltpu.SemaphoreType.DMA for _ in range(2 * NBUF)],
    )
    def k(table_hbm, idx_hbm, out_hbm, idx_v, *bufs_and_sems):
        bufs = bufs_and_sems[:NBUF]
        tbuf = bufs_and_sems[NBUF]
        gs = bufs_and_sems[NBUF + 1:2 * NBUF + 1]
        os = bufs_and_sems[2 * NBUF + 1:]
        wid = lax.axis_index("s") * NC + lax.axis_index("c")
        base = wid * b_per_w
        pltpu.sync_copy(idx_hbm.at[pl.ds(base, b_per_w)], idx_v)

        def gather(win, b):
            return pltpu.make_async_copy(
                table_hbm.at[idx_v.at[pl.ds(win * W, W)]], bufs[b], gs[b])

        def drain(win, b):
            return pltpu.make_async_copy(
                bufs[b], out_hbm.at[pl.ds(base + win * W, W)], os[b])

        gather(0, 0).start()

        @pl.loop(0, n_rounds)
        def _(r):
            for b in range(NBUF):
                w = NBUF * r + b
                gather(w, b).wait()
                drain(w, b).start()

                @pl.when(w >= NBUF - 1)
                def _():
                    drain(0, (b + 1) % NBUF).wait()

                @pl.when(w + 1 < n_full)
                def _():
                    gather(w + 1, (b + 1) % NBUF).start()

        t_off = n_full * W
        tgather = pltpu.make_async_copy(
            table_hbm.at[idx_v.at[pl.ds(t_off, tail)]], tbuf, gs[0])
        tgather.start()

        for b in range(NBUF - 1):
            drain(0, (n_full - (NBUF - 1) + b) % NBUF).wait()

        tgather.wait()
        pltpu.sync_copy(tbuf, out_hbm.at[pl.ds(base + t_off, tail)])

    return k(table, idx)


def kernel(x, embed_weight):
    b, t = x.shape
    d = embed_weight.shape[1]
    flat = x.T.reshape(b * t).astype(jnp.int32)
    out = _sc_gather(embed_weight, flat)
    return out.reshape(t, b, d).transpose(1, 0, 2)
